# Initial kernel scaffold; baseline (speedup 1.0000x reference)
import jax
import jax.numpy as jnp
from jax.experimental import pallas as pl


def kernel(x, edge_index, W, b):
    raise NotImplementedError("write your pallas kernel here")


# speedup vs baseline: 15.1219x; 15.1219x over previous
import functools

import jax
import jax.numpy as jnp
from jax import lax
from jax.experimental import pallas as pl
from jax.experimental.pallas import tpu as pltpu
from jax.experimental.pallas import tpu_sc as plsc

NC = 2
NS = 16
NW = NC * NS
CHUNK = 128
DEG_W = 16


def _ceil_to(a, m):
    return (a + m - 1) // m * m


def _tc_matmul(x, W):
    n, d_in = x.shape
    d_out = W.shape[0]
    bn = 1000

    def body(x_ref, w_ref, o_ref):
        o_ref[...] = lax.dot_general(
            x_ref[...], w_ref[...], (((1,), (1,)), ((), ())),
            preferred_element_type=jnp.float32,
            precision=lax.Precision.HIGHEST)

    return pl.pallas_call(
        body,
        grid=(n // bn,),
        in_specs=[pl.BlockSpec((bn, d_in), lambda i: (i, 0)),
                  pl.BlockSpec((d_out, d_in), lambda i: (0, 0))],
        out_specs=pl.BlockSpec((bn, d_out), lambda i: (i, 0)),
        out_shape=jax.ShapeDtypeStruct((n, d_out), jnp.float32),
    )(x, W)


def _tc_prescale(h, deg0, deg1):
    n, d = h.shape
    bn = 1000

    def body(h_ref, d0_ref, d1_ref, o_ref):
        deg = d0_ref[...][:, 0:1] + d1_ref[...][:, 0:1] + 1.0
        o_ref[...] = h_ref[...] * lax.rsqrt(deg)

    return pl.pallas_call(
        body,
        grid=(n // bn,),
        in_specs=[pl.BlockSpec((bn, d), lambda i: (i, 0)),
                  pl.BlockSpec((bn, DEG_W), lambda i: (i, 0)),
                  pl.BlockSpec((bn, DEG_W), lambda i: (i, 0))],
        out_specs=pl.BlockSpec((bn, d), lambda i: (i, 0)),
        out_shape=jax.ShapeDtypeStruct((n, d), jnp.float32),
    )(h, deg0, deg1)


def _tc_final(acc0, acc1, h, deg0, deg1, b):
    n, d = h.shape
    bn = 1000

    def body(a0_ref, a1_ref, h_ref, d0_ref, d1_ref, b_ref, o_ref):
        deg = d0_ref[...][:, 0:1] + d1_ref[...][:, 0:1] + 1.0
        dis = lax.rsqrt(deg)
        o_ref[...] = (dis * (a0_ref[...] + a1_ref[...])
                      + (dis * dis) * h_ref[...] + b_ref[...])

    return pl.pallas_call(
        body,
        grid=(n // bn,),
        in_specs=[pl.BlockSpec((bn, d), lambda i: (i, 0)),
                  pl.BlockSpec((bn, d), lambda i: (i, 0)),
                  pl.BlockSpec((bn, d), lambda i: (i, 0)),
                  pl.BlockSpec((bn, DEG_W), lambda i: (i, 0)),
                  pl.BlockSpec((bn, DEG_W), lambda i: (i, 0)),
                  pl.BlockSpec((1, d), lambda i: (0, 0))],
        out_specs=pl.BlockSpec((bn, d), lambda i: (i, 0)),
        out_shape=jax.ShapeDtypeStruct((n, d), jnp.float32),
    )(acc0, acc1, h, deg0, deg1, b.reshape(1, d))


def _sc_degree(dst_p, n_nodes):
    ep = dst_p.shape[0]
    cpw = ep // (NW * CHUNK)
    per_w = cpw * CHUNK
    per_sub = _ceil_to((n_nodes + 1 + NS - 1) // NS, CHUNK)
    acc_rows = per_sub * NS
    mesh = plsc.VectorSubcoreMesh(core_axis_name="c", subcore_axis_name="s")

    @functools.partial(
        pl.kernel, mesh=mesh,
        out_type=jax.ShapeDtypeStruct((NC, acc_rows, DEG_W), jnp.float32),
        scratch_types=[
            pltpu.VMEM((CHUNK,), jnp.int32),
            pltpu.VMEM((CHUNK, DEG_W), jnp.float32),
            pltpu.VMEM((CHUNK, DEG_W), jnp.float32),
            pltpu.VMEM_SHARED((acc_rows, DEG_W), jnp.float32),
        ])
    def deg_kernel(dst_hbm, out_hbm, idx_v, ones_v, zero_v, acc_sh):
        cid = lax.axis_index("c")
        sid = lax.axis_index("s")
        wid = cid * NS + sid

        @pl.loop(0, CHUNK)
        def _(r):
            ones_v.at[pl.ds(r, 1), pl.ds(0, DEG_W)][...] = jnp.ones(
                (1, DEG_W), jnp.float32)
            zero_v.at[pl.ds(r, 1), pl.ds(0, DEG_W)][...] = jnp.zeros(
                (1, DEG_W), jnp.float32)

        @pl.loop(0, per_sub // CHUNK)
        def _(i):
            pltpu.sync_copy(zero_v,
                            acc_sh.at[pl.ds(sid * per_sub + i * CHUNK, CHUNK)])

        plsc.subcore_barrier()

        @pl.loop(0, cpw)
        def _(c):
            pltpu.sync_copy(dst_hbm.at[pl.ds(wid * per_w + c * CHUNK, CHUNK)],
                            idx_v)
            pltpu.sync_copy(ones_v, acc_sh.at[idx_v], add=True)

        plsc.subcore_barrier()
        pltpu.sync_copy(acc_sh.at[pl.ds(sid * per_sub, per_sub)],
                        out_hbm.at[cid, pl.ds(sid * per_sub, per_sub)])

    return deg_kernel(dst_p), acc_rows


def _sc_aggregate(hs, src_p, dst_p, n_nodes):
    ep = src_p.shape[0]
    d = hs.shape[1]
    cpw = ep // (NW * CHUNK)
    per_w = cpw * CHUNK
    per_sub = _ceil_to((n_nodes + 1 + NS - 1) // NS, CHUNK)
    acc_rows = per_sub * NS
    mesh = plsc.VectorSubcoreMesh(core_axis_name="c", subcore_axis_name="s")

    @functools.partial(
        pl.kernel, mesh=mesh,
        out_type=jax.ShapeDtypeStruct((NC, acc_rows, d), jnp.float32),
        scratch_types=[
            pltpu.VMEM((CHUNK,), jnp.int32),
            pltpu.VMEM((CHUNK,), jnp.int32),
            pltpu.VMEM((CHUNK, d), jnp.float32),
            pltpu.VMEM_SHARED((acc_rows, d), jnp.float32),
            pltpu.SemaphoreType.DMA,
        ])
    def agg_kernel(hs_hbm, src_hbm, dst_hbm, out_hbm,
                   src_v, dst_v, rows_v, acc_sh, sem):
        cid = lax.axis_index("c")
        sid = lax.axis_index("s")
        wid = cid * NS + sid

        @pl.loop(0, CHUNK)
        def _(r):
            @pl.loop(0, d // 16)
            def _(c16):
                rows_v.at[pl.ds(r, 1), pl.ds(c16 * 16, 16)][...] = jnp.zeros(
                    (1, 16), jnp.float32)

        @pl.loop(0, per_sub // CHUNK)
        def _(i):
            pltpu.sync_copy(rows_v,
                            acc_sh.at[pl.ds(sid * per_sub + i * CHUNK, CHUNK)])

        plsc.subcore_barrier()

        @pl.loop(0, cpw)
        def _(c):
            base = wid * per_w + c * CHUNK
            pltpu.sync_copy(src_hbm.at[pl.ds(base, CHUNK)], src_v)
            pltpu.sync_copy(dst_hbm.at[pl.ds(base, CHUNK)], dst_v)
            pltpu.async_copy(hs_hbm.at[src_v], rows_v, sem).wait()
            pltpu.sync_copy(rows_v, acc_sh.at[dst_v], add=True)

        plsc.subcore_barrier()
        pltpu.sync_copy(acc_sh.at[pl.ds(sid * per_sub, per_sub)],
                        out_hbm.at[cid, pl.ds(sid * per_sub, per_sub)])

    return agg_kernel(hs, src_p, dst_p)


def kernel(x, edge_index, W, b):
    n, _ = x.shape
    e = edge_index.shape[1]
    src = edge_index[0].astype(jnp.int32)
    dst = edge_index[1].astype(jnp.int32)

    ep = _ceil_to(e, NW * CHUNK)
    src_p = jnp.concatenate([src, jnp.zeros((ep - e,), jnp.int32)])
    dst_p = jnp.concatenate([dst, jnp.full((ep - e,), n, jnp.int32)])

    deg_part, _ = _sc_degree(dst_p, n)
    h = _tc_matmul(x, W)
    d0 = deg_part[0, :n]
    d1 = deg_part[1, :n]
    hs = _tc_prescale(h, d0, d1)
    acc_part = _sc_aggregate(hs, src_p, dst_p, n)
    out = _tc_final(acc_part[0, :n], acc_part[1, :n], h, d0, d1, b)
    return out
